# baseline (speedup 1.0000x reference)
import functools

import jax
import jax.numpy as jnp
from jax import lax
from jax.experimental import pallas as pl
from jax.experimental.pallas import tpu as pltpu
from jax.experimental.pallas import tpu_sc as plsc

NUM_ROWS = 1000000
DIM = 32

B_TOTAL = 16384 * 200
NSPLIT = 2
B_HALF = B_TOTAL // NSPLIT
G = 128
CHUNK = 1024
KJ = CHUNK // G
NW = 32
B_PER_W = B_HALF // NW
CHUNKS_PER_W = B_PER_W // CHUNK
NPAIR = CHUNKS_PER_W // 2


def _body(idx_hbm, w_hbm, out_hbm, idx_v, rows_v, sem_g, sem_w0, sem_w1):
    wid = lax.axis_index("s") * 2 + lax.axis_index("c")
    base = wid * B_PER_W
    sem_w = (sem_w0, sem_w1)

    def stage_idx(c, p):
        pltpu.sync_copy(idx_hbm.at[pl.ds(base + c * CHUNK, CHUNK)],
                        idx_v.at[p])

    def issue_gathers(p):
        for j in range(KJ):
            pltpu.async_copy(
                w_hbm.at[idx_v.at[p, pl.ds(j * G, G)]],
                rows_v.at[p, pl.ds(j * G, G), :],
                sem_g,
            )

    def drain_gathers(p):
        pltpu.make_async_copy(
            w_hbm.at[pl.ds(0, CHUNK), :], rows_v.at[p], sem_g
        ).wait()

    def issue_writeback(c, p):
        pltpu.async_copy(
            rows_v.at[p],
            out_hbm.at[pl.ds(base + c * CHUNK, CHUNK), pl.ds(0, DIM)],
            sem_w[p],
        )

    def drain_writeback(p):
        pltpu.make_async_copy(
            w_hbm.at[pl.ds(0, CHUNK), :], rows_v.at[p], sem_w[p]
        ).wait()

    stage_idx(0, 0)
    issue_gathers(0)

    def pair(cc, carry):
        c0 = 2 * cc

        drain_gathers(0)
        issue_writeback(c0, 0)
        stage_idx(c0 + 1, 1)

        @pl.when(cc > 0)
        def _():
            drain_writeback(1)
        issue_gathers(1)

        drain_gathers(1)
        issue_writeback(c0 + 1, 1)

        @pl.when(cc < NPAIR - 1)
        def _():
            stage_idx(c0 + 2, 0)
            drain_writeback(0)
            issue_gathers(0)
        return carry

    lax.fori_loop(0, NPAIR, pair, 0)
    drain_writeback(0)
    drain_writeback(1)


_mesh = plsc.VectorSubcoreMesh(core_axis_name="c", subcore_axis_name="s")

_gather = functools.partial(
    pl.kernel,
    out_type=jax.ShapeDtypeStruct((B_HALF, 128), jnp.float32),
    mesh=_mesh,
    scratch_types=[
        pltpu.VMEM((2, CHUNK), jnp.int32),
        pltpu.VMEM((2, CHUNK, DIM), jnp.float32),
        pltpu.SemaphoreType.DMA,
        pltpu.SemaphoreType.DMA,
        pltpu.SemaphoreType.DMA,
    ],
    compiler_params=pltpu.CompilerParams(use_tc_tiling_on_sc=False),
)(_body)


def kernel(x, weight):
    idx = x.reshape(B_TOTAL).astype(jnp.int32)
    rows = x.shape[0] // NSPLIT
    halves = [
        _gather(idx[i * B_HALF:(i + 1) * B_HALF], weight)[:, :DIM]
        .reshape(rows, x.shape[1], DIM)
        for i in range(NSPLIT)
    ]
    return jnp.concatenate(halves, axis=0)


# speedup vs baseline: 1.1980x; 1.1980x over previous
import functools

import jax
import jax.numpy as jnp
from jax import lax
from jax.experimental import pallas as pl
from jax.experimental.pallas import tpu as pltpu
from jax.experimental.pallas import tpu_sc as plsc

NUM_ROWS = 1000000
DIM = 32

B_TOTAL = 16384 * 200
G = 128
CHUNK = 1024
KJ = CHUNK // G
NW = 32
B_PER_W = B_TOTAL // NW
CHUNKS_PER_W = B_PER_W // CHUNK
NPAIR = CHUNKS_PER_W // 2


def _body(idx_hbm, w_hbm, out_hbm, idx_v, rows_v, sem_g, sem_w0, sem_w1):
    wid = lax.axis_index("s") * 2 + lax.axis_index("c")
    base = wid * B_PER_W
    sem_w = (sem_w0, sem_w1)

    def stage_idx(c, p):
        pltpu.sync_copy(idx_hbm.at[pl.ds(base + c * CHUNK, CHUNK)],
                        idx_v.at[p])

    def issue_gathers(p):
        for j in range(KJ):
            pltpu.async_copy(
                w_hbm.at[idx_v.at[p, pl.ds(j * G, G)]],
                rows_v.at[p, pl.ds(j * G, G), :],
                sem_g,
            )

    def drain_gathers(p):
        pltpu.make_async_copy(
            w_hbm.at[pl.ds(0, CHUNK), :], rows_v.at[p], sem_g
        ).wait()

    def issue_writeback(c, p):
        pltpu.async_copy(
            rows_v.at[p],
            out_hbm.at[pl.ds(base + c * CHUNK, CHUNK), pl.ds(0, DIM)],
            sem_w[p],
        )

    def drain_writeback(p):
        pltpu.make_async_copy(
            w_hbm.at[pl.ds(0, CHUNK), :], rows_v.at[p], sem_w[p]
        ).wait()

    stage_idx(0, 0)
    issue_gathers(0)

    def pair(cc, carry):
        c0 = 2 * cc

        drain_gathers(0)
        issue_writeback(c0, 0)
        stage_idx(c0 + 1, 1)

        @pl.when(cc > 0)
        def _():
            drain_writeback(1)
        issue_gathers(1)

        drain_gathers(1)
        issue_writeback(c0 + 1, 1)

        @pl.when(cc < NPAIR - 1)
        def _():
            stage_idx(c0 + 2, 0)
            drain_writeback(0)
            issue_gathers(0)
        return carry

    lax.fori_loop(0, NPAIR, pair, 0)
    drain_writeback(0)
    drain_writeback(1)


_mesh = plsc.VectorSubcoreMesh(core_axis_name="c", subcore_axis_name="s")

_gather = functools.partial(
    pl.kernel,
    out_type=jax.ShapeDtypeStruct((B_TOTAL, 128), jnp.float32),
    mesh=_mesh,
    scratch_types=[
        pltpu.VMEM((2, CHUNK), jnp.int32),
        pltpu.VMEM((2, CHUNK, DIM), jnp.float32),
        pltpu.SemaphoreType.DMA,
        pltpu.SemaphoreType.DMA,
        pltpu.SemaphoreType.DMA,
    ],
    compiler_params=pltpu.CompilerParams(use_tc_tiling_on_sc=False),
)(_body)


def kernel(x, weight):
    idx = x.reshape(B_TOTAL).astype(jnp.int32)
    out = _gather(idx, weight)
    return out[:, :DIM].reshape(*x.shape, DIM)
